# baseline (speedup 1.0000x reference)
import functools
import math

import jax
import jax.numpy as jnp
from jax import lax
from jax.experimental import pallas as pl
from jax.experimental.pallas import tpu as pltpu
from jax.experimental.pallas import tpu_sc as plsc

D_MODEL = 64
SCALE = math.sqrt(D_MODEL)

NC = 2
NS = 16
NW = NC * NS
LANES = 16

BATCH = 128
NBUF = 4


def _scale_batch(src, dst, b):
    @functools.partial(plsc.parallel_loop, 0, BATCH, unroll=8)
    def _rows(r):
        for c in range(D_MODEL // LANES):
            sl = pl.ds(c * LANES, LANES)
            dst[b, r, sl] = src[b, r, sl] * SCALE


def _emb_body(nbatch, b_per_w, x_hbm, lut_hbm, out_hbm, idx_v, rows_g,
              rows_s, *sems):
    gsems = sems[:NBUF]
    ssems = sems[NBUF:]
    wid = lax.axis_index("s") * NC + lax.axis_index("c")
    base = wid * b_per_w

    pltpu.sync_copy(x_hbm.at[wid], idx_v)

    def start_gather(j, b):
        pltpu.async_copy(lut_hbm.at[idx_v.at[j]], rows_g.at[b], gsems[b])

    def wait_gather(j, b):
        pltpu.make_async_copy(
            lut_hbm.at[idx_v.at[j]], rows_g.at[b], gsems[b]).wait()

    def out_slice(j):
        return out_hbm.at[pl.ds(base + j * BATCH, BATCH)]

    def start_scatter(j, b):
        pltpu.async_copy(rows_s.at[b], out_slice(j), ssems[b])

    def wait_scatter(j, b):
        pltpu.make_async_copy(rows_s.at[b], out_slice(j), ssems[b]).wait()

    for b in range(NBUF):
        start_gather(b, b)

    for b in range(NBUF):
        wait_gather(b, b)
        _scale_batch(rows_g, rows_s, b)
        start_scatter(b, b)
        start_gather(b + NBUF, b)

    steady = nbatch - NBUF

    @pl.loop(NBUF, steady, step=NBUF)
    def _main(g):
        for b in range(NBUF):
            j = g + b
            wait_gather(j, b)
            wait_scatter(j - NBUF, b)
            _scale_batch(rows_g, rows_s, b)
            start_scatter(j, b)
            start_gather(j + NBUF, b)

    for b in range(NBUF):
        j = steady + b
        wait_gather(j, b)
        wait_scatter(j - NBUF, b)
        _scale_batch(rows_g, rows_s, b)
        start_scatter(j, b)
    for b in range(NBUF):
        wait_scatter(steady + b, b)


def kernel(x, lut):
    rows, cols = x.shape
    total = rows * cols
    assert total % (NW * BATCH) == 0
    b_per_w = total // NW
    nbatch = b_per_w // BATCH

    x_flat = x.reshape(NW, nbatch, BATCH)

    mesh = plsc.VectorSubcoreMesh(
        core_axis_name="c", subcore_axis_name="s",
        num_cores=NC, num_subcores=NS)

    run = pl.kernel(
        functools.partial(_emb_body, nbatch, b_per_w),
        out_type=jax.ShapeDtypeStruct((total, D_MODEL), jnp.float32),
        mesh=mesh,
        scratch_types=(
            [pltpu.VMEM((nbatch, BATCH), jnp.int32),
             pltpu.VMEM((NBUF, BATCH, D_MODEL), jnp.float32),
             pltpu.VMEM((NBUF, BATCH, D_MODEL), jnp.float32)]
            + [pltpu.SemaphoreType.DMA] * (2 * NBUF)
        ),
        compiler_params=pltpu.CompilerParams(use_tc_tiling_on_sc=False),
    )
    out = run(x_flat, lut)
    return out.reshape(rows, cols, D_MODEL)
